# baseline (speedup 1.0000x reference)
import jax
import jax.numpy as jnp
from jax.experimental import pallas as pl

_ND = 2


def _body(ids_ref, x_ref, out_ref):
    w = (ids_ref[0] != 0).astype(jnp.float32)
    s = jax.lax.dot_general(
        w, x_ref[0], (((1,), (0,)), ((), ())),
        preferred_element_type=jnp.float32)
    cnt = jnp.sum(w)
    out_ref[0] = s / cnt


def kernel(x, input):
    B, S, D = x.shape
    dc = D // _ND
    ids3 = input.reshape(B, 1, S).astype(jnp.int32)
    out = pl.pallas_call(
        _body,
        grid=(B, _ND),
        in_specs=[
            pl.BlockSpec((1, 1, S), lambda b, j: (b, 0, 0)),
            pl.BlockSpec((1, S, dc), lambda b, j: (b, 0, j)),
        ],
        out_specs=pl.BlockSpec((1, 1, dc), lambda b, j: (b, 0, j)),
        out_shape=jax.ShapeDtypeStruct((B, 1, D), jnp.float32),
    )(ids3, x)
    return out.reshape(B, D)


# speedup vs baseline: 1.1039x; 1.1039x over previous
import jax
import jax.numpy as jnp
from jax.experimental import pallas as pl
from jax.experimental.pallas import tpu as pltpu

_CH = 512
_NBUF = 4


def _body(ids_ref, x_hbm, out_ref, buf, sem):
    B = out_ref.shape[0]
    S = ids_ref.shape[2]
    npc = S // _CH
    total = B * npc
    dn = (((1,), (0,)), ((), ()))

    def start(i):
        b, c = divmod(i, npc)
        pltpu.make_async_copy(
            x_hbm.at[b, pl.ds(c * _CH, _CH), :],
            buf.at[i % _NBUF], sem.at[i % _NBUF]).start()

    def wait(i):
        b, c = divmod(i, npc)
        pltpu.make_async_copy(
            x_hbm.at[b, pl.ds(c * _CH, _CH), :],
            buf.at[i % _NBUF], sem.at[i % _NBUF]).wait()

    for k in range(_NBUF - 1):
        start(k)

    for b in range(B):
        w = (ids_ref[b] != 0).astype(jnp.float32)
        acc = None
        for c in range(npc):
            i = b * npc + c
            if i + _NBUF - 1 < total:
                start(i + _NBUF - 1)
            wait(i)
            part = jax.lax.dot_general(
                w[:, c * _CH:(c + 1) * _CH], buf[i % _NBUF], dn,
                preferred_element_type=jnp.float32)
            acc = part if acc is None else acc + part
        cnt = jnp.sum(w)
        out_ref[b] = acc / cnt


def kernel(x, input):
    B, S, D = x.shape
    ids3 = input.reshape(B, 1, S).astype(jnp.int32)
    out = pl.pallas_call(
        _body,
        grid=(1,),
        in_specs=[
            pl.BlockSpec((B, 1, S), lambda i: (0, 0, 0)),
            pl.BlockSpec(memory_space=pl.ANY),
        ],
        out_specs=pl.BlockSpec((B, 1, D), lambda i: (0, 0, 0)),
        out_shape=jax.ShapeDtypeStruct((B, 1, D), jnp.float32),
        scratch_shapes=[
            pltpu.VMEM((_NBUF, _CH, D), jnp.float32),
            pltpu.SemaphoreType.DMA((_NBUF,)),
        ],
    )(ids3, x)
    return out.reshape(B, D)


# speedup vs baseline: 1.1155x; 1.0105x over previous
import jax
import jax.numpy as jnp
from jax.experimental import pallas as pl


def _body(ids_ref, x_ref, out_ref):
    w = (ids_ref[0] != 0).astype(jnp.float32)
    s = jax.lax.dot_general(
        w, x_ref[0], (((1,), (0,)), ((), ())),
        preferred_element_type=jnp.float32)
    cnt = jnp.sum(w)
    out_ref[0] = s / cnt


def kernel(x, input):
    B, S, D = x.shape
    ids3 = input.reshape(B, 1, S).astype(jnp.int32)
    out = pl.pallas_call(
        _body,
        grid=(B,),
        in_specs=[
            pl.BlockSpec((1, 1, S), lambda b: (b, 0, 0)),
            pl.BlockSpec((1, S, D), lambda b: (b, 0, 0)),
        ],
        out_specs=pl.BlockSpec((1, 1, D), lambda b: (b, 0, 0)),
        out_shape=jax.ShapeDtypeStruct((B, 1, D), jnp.float32),
    )(ids3, x)
    return out.reshape(B, D)
